# baseline (speedup 1.0000x reference)
import functools

import jax
import jax.numpy as jnp
from jax import lax
from jax.experimental import pallas as pl
from jax.experimental.pallas import tpu as pltpu
from jax.experimental.pallas import tpu_sc as plsc

BATCH = 16384
SEQ = 26
D = 32
NW = 32
PER_B = BATCH // NW
GB = 8
NCH = PER_B // GB
T4_ROWS = 250000
RCH = 64
RCH_U = RCH // 8
K0_LOOP = 62

_MESH = plsc.VectorSubcoreMesh(core_axis_name="c", subcore_axis_name="s")


def _repack(table):
    @functools.partial(
        pl.kernel,
        mesh=_MESH,
        out_type=jax.ShapeDtypeStruct((T4_ROWS, 128), jnp.float32),
        scratch_types=[
            pltpu.VMEM((4 * RCH, D), jnp.float32),
            pltpu.VMEM((4 * RCH, D), jnp.float32),
            pltpu.VMEM((RCH, 128), jnp.float32),
            pltpu.VMEM((RCH, 128), jnp.float32),
            pltpu.SemaphoreType.DMA,
            pltpu.SemaphoreType.DMA,
        ],
        compiler_params=pltpu.CompilerParams(use_tc_tiling_on_sc=True),
    )
    def k(table_hbm, t4_hbm, in_a, in_b, pk_a, pk_b, sem_a, sem_b):
        wid = lax.axis_index("s") * 2 + lax.axis_index("c")
        base_u = wid * 976 + jnp.minimum(wid, 18)
        n_u = 976 + jnp.where(wid < 18, 1, 0)
        nch = (n_u + RCH_U - 1) // RCH_U
        ins = (in_a, in_b)
        pks = (pk_a, pk_b)
        sems = (sem_a, sem_b)

        def start_q(c, p):
            return 8 * jnp.minimum(base_u + c * RCH_U, base_u + n_u - RCH_U)

        def fire(c, p):
            q = start_q(c, p)
            pltpu.async_copy(
                table_hbm.at[pl.ds(4 * q, 4 * RCH)], ins[p], sems[p]
            )

        def process(c, p):
            q = start_q(c, p)
            pltpu.make_async_copy(
                table_hbm.at[pl.ds(4 * q, 4 * RCH)], ins[p], sems[p]
            ).wait()
            src, dst = ins[p], pks[p]
            for r in range(RCH):
                for a in range(4):
                    for h in range(2):
                        dst[r, pl.ds(32 * a + 16 * h, 16)] = src[
                            4 * r + a, pl.ds(16 * h, 16)
                        ]
            pltpu.sync_copy(dst, t4_hbm.at[pl.ds(q, RCH)])

        fire(0, 0)

        def body(c, carry):
            return carry

        def loop_body(g, carry):
            c = 2 * g

            @pl.when(c < nch)
            def _():
                fire(c + 1, 1)
                process(c, 0)

            @pl.when(c + 1 < nch)
            def _():
                fire(c + 2, 0)
                process(c + 1, 1)

            return carry

        lax.fori_loop(0, K0_LOOP, loop_body, 0)

    return k(table)


def _gather_call(idx4, off, table4):
    @functools.partial(
        pl.kernel,
        mesh=_MESH,
        out_type=jax.ShapeDtypeStruct((BATCH, SEQ, D), jnp.float32),
        scratch_types=[
            pltpu.VMEM((PER_B, SEQ), jnp.int32),
            pltpu.VMEM((PER_B, SEQ), jnp.int32),
            pltpu.VMEM((GB, SEQ, 128), jnp.float32),
            pltpu.VMEM((GB, SEQ, 128), jnp.float32),
            pltpu.VMEM((GB, SEQ, D), jnp.float32),
            pltpu.VMEM((GB, SEQ, D), jnp.float32),
            pltpu.SemaphoreType.DMA,
            pltpu.SemaphoreType.DMA,
        ],
        compiler_params=pltpu.CompilerParams(use_tc_tiling_on_sc=False),
    )
    def k(idx4_hbm, off_hbm, t4_hbm, out_hbm,
          idx4_v, off_v, gbuf_a, gbuf_b, obuf_a, obuf_b, sem_a, sem_b):
        wid = lax.axis_index("s") * 2 + lax.axis_index("c")
        b0 = wid * PER_B
        pltpu.sync_copy(idx4_hbm.at[pl.ds(b0, PER_B)], idx4_v)
        pltpu.sync_copy(off_hbm.at[pl.ds(b0, PER_B)], off_v)

        gbufs = (gbuf_a, gbuf_b)
        obufs = (obuf_a, obuf_b)
        sems = (sem_a, sem_b)

        def fire(c, p):
            for i in range(GB):
                pltpu.async_copy(
                    t4_hbm.at[idx4_v.at[c * GB + i]], gbufs[p].at[i], sems[p]
                )

        def process(c, p):
            gbuf, obuf, sem = gbufs[p], obufs[p], sems[p]
            for i in range(GB):
                pltpu.make_async_copy(
                    t4_hbm.at[idx4_v.at[c * GB + i]], gbuf.at[i], sem
                ).wait()
            for i in range(GB):
                r0 = off_v[c * GB + i, pl.ds(0, 16)]
                r1 = off_v[c * GB + i, pl.ds(SEQ - 16, 16)]
                for s in range(SEQ):
                    o = r0[s] if s < 16 else r1[s - (SEQ - 16)]
                    obuf[i, s, pl.ds(0, 16)] = gbuf[i, s, pl.ds(o, 16)]
                    obuf[i, s, pl.ds(16, 16)] = gbuf[i, s, pl.ds(o + 16, 16)]
            pltpu.sync_copy(obuf, out_hbm.at[pl.ds(b0 + c * GB, GB)])

        fire(0, 0)

        def body(g, carry):
            c = 2 * g
            fire(c + 1, 1)
            process(c, 0)
            fire(c + 2, 0)
            process(c + 1, 1)
            return carry

        lax.fori_loop(0, NCH // 2 - 1, body, 0)
        c = NCH - 2
        fire(c + 1, 1)
        process(c, 0)
        process(c + 1, 1)

    return k(idx4, off, table4)


def kernel(inputs, embeddings):
    idx = inputs.astype(jnp.int32)
    idx4 = idx >> 2
    off = (idx & 3) << 5
    t4 = _repack(embeddings[:1000000])
    return _gather_call(idx4, off, t4)


# speedup vs baseline: 1.2143x; 1.2143x over previous
import functools

import jax
import jax.numpy as jnp
from jax import lax
from jax.experimental import pallas as pl
from jax.experimental.pallas import tpu as pltpu
from jax.experimental.pallas import tpu_sc as plsc

BATCH = 16384
SEQ = 26
D = 32
NW = 32
PER_B = BATCH // NW
NB = 32
NCH = PER_B // NB


def _gather_call(idx, table):
    mesh = plsc.VectorSubcoreMesh(core_axis_name="c", subcore_axis_name="s")

    @functools.partial(
        pl.kernel,
        mesh=mesh,
        out_type=jax.ShapeDtypeStruct((BATCH, SEQ, D), jnp.float32),
        scratch_types=[
            pltpu.VMEM((PER_B, SEQ), jnp.int32),
            pltpu.VMEM((NB, SEQ, D), jnp.float32),
            pltpu.VMEM((NB, SEQ, D), jnp.float32),
            pltpu.SemaphoreType.DMA,
            pltpu.SemaphoreType.DMA,
        ],
        compiler_params=pltpu.CompilerParams(use_tc_tiling_on_sc=False),
    )
    def k(idx_hbm, table_hbm, out_hbm, idx_v, buf_a, buf_b, sem_a, sem_b):
        wid = lax.axis_index("s") * 2 + lax.axis_index("c")
        b0 = wid * PER_B
        pltpu.sync_copy(idx_hbm.at[pl.ds(b0, PER_B)], idx_v)

        bufs = (buf_a, buf_b)
        sems = (sem_a, sem_b)

        def fire(c, buf, sem):
            def body(i, carry):
                pltpu.async_copy(
                    table_hbm.at[idx_v.at[c * NB + i]], buf.at[i], sem
                )
                return carry

            lax.fori_loop(0, NB, body, 0)

        fire(0, bufs[0], sems[0])
        for c in range(NCH):
            buf, sem = bufs[c % 2], sems[c % 2]
            if c + 1 < NCH:
                fire(c + 1, bufs[(c + 1) % 2], sems[(c + 1) % 2])
            dst = out_hbm.at[pl.ds(b0 + c * NB, NB)]
            pltpu.make_async_copy(dst, buf, sem).wait()
            pltpu.sync_copy(buf, dst)

    return k(idx, table)


def kernel(inputs, embeddings):
    return _gather_call(inputs.astype(jnp.int32), embeddings)
